# baseline (speedup 1.0000x reference)
import functools

import jax
import jax.numpy as jnp
from jax import lax
from jax.experimental import pallas as pl
from jax.experimental.pallas import tpu as pltpu
from jax.experimental.pallas import tpu_sc as plsc

EPSN = 1e-9

_N = 10000
_D = 128
_E = 320000
_NP = 10240
_CHUNK = 128
_NCHUNKS = 2560
_EP = _NCHUNKS * _CHUNK
_H = _D // 2
_ROWS_PER_TILE = _NP // 16
_CHUNKS_C0 = 104
_CHUNKS_C1 = 56
_CHUNKS_MAX = 104


def _sc_mesh():
    return plsc.VectorSubcoreMesh(core_axis_name="c", subcore_axis_name="s")


_HCHUNK = 2000


def _tc_degree_body(src_ref, deg_ref):
    i = pl.program_id(0)

    @pl.when(i == 0)
    def _():
        deg_ref[...] = jnp.zeros_like(deg_ref)

    s = src_ref[0, 0, :]
    hi = jax.lax.shift_right_logical(s, 7)
    lo = jax.lax.bitwise_and(s, 127)
    r = lax.broadcasted_iota(jnp.int32, (_D, _HCHUNK), 0)
    oh_hi_t = (r == hi[None, :]).astype(jnp.bfloat16)
    c = lax.broadcasted_iota(jnp.int32, (_HCHUNK, _D), 1)
    oh_lo = (c == lo[:, None]).astype(jnp.bfloat16)
    deg_ref[...] += jnp.dot(oh_hi_t, oh_lo, preferred_element_type=jnp.float32)


def _tc_degree(src):
    grid = (_E // _HCHUNK,)
    return pl.pallas_call(
        _tc_degree_body,
        grid=grid,
        in_specs=[pl.BlockSpec((1, 1, _HCHUNK), lambda i: (i, 0, 0))],
        out_specs=pl.BlockSpec((_D, _D), lambda i: (0, 0)),
        out_shape=jax.ShapeDtypeStruct((_D, _D), jnp.float32),
    )(src.reshape(_E // _HCHUNK, 1, _HCHUNK))


@functools.cache
def _sc_segsum_kernel():
    return functools.partial(
        pl.kernel,
        mesh=_sc_mesh(),
        out_type=jax.ShapeDtypeStruct((2 * _NP, _D), jnp.float32),
        scratch_types=[
            pltpu.VMEM((_CHUNKS_MAX, _CHUNK), jnp.int32),
            pltpu.VMEM((_CHUNK,), jnp.int32),
            pltpu.VMEM((_CHUNK,), jnp.int32),
            pltpu.VMEM((_CHUNK, _D), jnp.float32),
            pltpu.VMEM((_CHUNK, _D), jnp.float32),
            pltpu.VMEM_SHARED((_NP, _D), jnp.float32),
            pltpu.SemaphoreType.DMA,
            pltpu.SemaphoreType.DMA,
            pltpu.SemaphoreType.DMA,
            pltpu.SemaphoreType.DMA,
            pltpu.SemaphoreType.DMA,
        ],
    )(_sc_segsum_body)


def _sc_segsum_body(src_hbm, dst_hbm, y_hbm, zeros_hbm, out_hbm,
                    srci_all, dsti0, dsti1, rows0, rows1, acc_sp,
                    sem0, sem1, semi0, semi1, semp):
    c = lax.axis_index("c")
    s = lax.axis_index("s")
    n = jnp.where(c == 0, _CHUNKS_C0, _CHUNKS_C1)
    base = pl.multiple_of(
        jnp.where(c == 0, s * _CHUNKS_C0, 16 * _CHUNKS_C0 + s * _CHUNKS_C1), 8)

    @pl.when(c == 0)
    def _ld0():
        pltpu.async_copy(src_hbm.at[pl.ds(base, _CHUNKS_C0)],
                         srci_all.at[pl.ds(0, _CHUNKS_C0)], semp)

    @pl.when(c == 1)
    def _ld1():
        pltpu.async_copy(src_hbm.at[pl.ds(base, _CHUNKS_C1)],
                         srci_all.at[pl.ds(0, _CHUNKS_C1)], semp)

    stripe = pl.ds(s * _ROWS_PER_TILE, _ROWS_PER_TILE)
    pltpu.sync_copy(zeros_hbm.at[stripe], acc_sp.at[stripe])

    @pl.when(c == 0)
    def _wt0():
        pltpu.make_async_copy(src_hbm.at[pl.ds(base, _CHUNKS_C0)],
                              srci_all.at[pl.ds(0, _CHUNKS_C0)], semp).wait()

    @pl.when(c == 1)
    def _wt1():
        pltpu.make_async_copy(src_hbm.at[pl.ds(base, _CHUNKS_C1)],
                              srci_all.at[pl.ds(0, _CHUNKS_C1)], semp).wait()

    plsc.subcore_barrier()

    rows = (rows0, rows1)
    dsti = (dsti0, dsti1)
    sems = (sem0, sem1)
    semi = (semi0, semi1)

    def idx_load(j, b):
        off = (base + j) * _CHUNK
        pltpu.async_copy(dst_hbm.at[pl.ds(off, _CHUNK)], dsti[b], semi[b])

    def start_gather(b):
        pltpu.make_async_copy(dst_hbm.at[pl.ds(0, _CHUNK)], dsti[b],
                              semi[b]).wait()
        pltpu.async_copy(y_hbm.at[dsti[b]], rows[b], sems[b])

    def finish_chunk(j, b):
        pltpu.make_async_copy(y_hbm.at[dsti[b]], rows[b], sems[b]).wait()

        @pl.when(j + 2 < n)
        def _():
            idx_load(j + 2, b)

        pltpu.sync_copy(rows[b], acc_sp.at[srci_all.at[j]], add=True)

    idx_load(0, 0)
    idx_load(1, 1)
    start_gather(0)
    npairs = n // 2

    def body(g, carry):
        start_gather(1)
        finish_chunk(g * 2, 0)

        @pl.when(g < npairs - 1)
        def _():
            start_gather(0)

        finish_chunk(g * 2 + 1, 1)
        return carry

    lax.fori_loop(0, npairs, body, 0)
    plsc.subcore_barrier()

    pltpu.sync_copy(acc_sp.at[stripe],
                    out_hbm.at[pl.ds(c * _NP + s * _ROWS_PER_TILE, _ROWS_PER_TILE)])


def _tc_scale_body(x_ref, deg_ref, y_ref):
    deg = deg_ref[...]
    dis = jnp.where(deg > 0, lax.rsqrt(deg), 0.0)
    y_ref[...] = x_ref[...] * dis


def _tc_scale(x_pad, deg_col):
    return pl.pallas_call(
        _tc_scale_body,
        out_shape=jax.ShapeDtypeStruct((_NP, _D), jnp.float32),
    )(x_pad, deg_col)


def _acosh(t):
    return jnp.log(t + jnp.sqrt((t - 1.0) * (t + 1.0)))


def _tc_dense_body(eps_ref, x_ref, sa_ref, sb_ref, w1_ref, b1_ref, w2_ref, b2_ref,
                   o_ref):
    R = x_ref.shape[0]
    col = lax.broadcasted_iota(jnp.int32, (R, _D), 1)
    m = jnp.where(col > 0, 1.0, 0.0)
    e0 = jnp.where(col == 0, 1.0, 0.0)

    def logmap0_s(z):
        zs = z * m
        xn = jnp.sqrt(jnp.clip(jnp.sum(zs * zs, axis=-1, keepdims=True), EPSN, None))
        t = jnp.clip(z[:, :1], 1.0 + 1e-7, None)
        return (_acosh(t) / xn) * zs

    def sinh_cosh(n):
        en = jnp.exp(n)
        inv = 1.0 / en
        return 0.5 * (en - inv), 0.5 * (en + inv)

    def exp_proj(v):
        n = jnp.sqrt(jnp.clip(jnp.sum(v * v, axis=-1, keepdims=True), EPSN, None))
        sh, _ = sinh_cosh(n)
        q = (sh / n) * v
        t = jnp.sqrt(1.0 + jnp.sum(q * q, axis=-1, keepdims=True))
        return q + e0 * t

    eps = eps_ref[0, 0]
    x = x_ref[...]
    S = sa_ref[...] + sb_ref[...]

    sumsq = jnp.sum(S * S, axis=-1, keepdims=True)
    inner = sumsq - 2.0 * (S[:, :1] * S[:, :1])
    denom = jnp.sqrt(jnp.clip(-inner, EPSN, None))
    h = S / denom

    v = (1.0 + eps) * logmap0_s(x) + logmap0_s(h)
    z = exp_proj(v)

    def layer(z, w_ref, b_ref):
        u = logmap0_s(z)
        o = jnp.dot(u, w_ref[...], preferred_element_type=jnp.float32) + b_ref[...]
        z1 = exp_proj(o)
        ua = jnp.tanh(logmap0_s(z1))
        n = jnp.sqrt(jnp.clip(jnp.sum(ua * ua, axis=-1, keepdims=True), EPSN, None))
        sh, ch = sinh_cosh(n)
        return (sh / n) * ua + e0 * ch

    z = layer(z, w1_ref, b1_ref)
    z = layer(z, w2_ref, b2_ref)
    o_ref[...] = z


def _tc_dense(x, Sa, Sb, eps, Wt1, bt1, Wt2, bt2):
    R = 1000
    grid = (_N // R,)
    row_spec = pl.BlockSpec((R, _D), lambda i: (i, 0))
    full_spec = pl.BlockSpec((_D, _D), lambda i: (0, 0))
    bias_spec = pl.BlockSpec((1, _D), lambda i: (0, 0))
    return pl.pallas_call(
        _tc_dense_body,
        grid=grid,
        in_specs=[
            pl.BlockSpec(memory_space=pltpu.SMEM),
            row_spec, row_spec, row_spec, full_spec, bias_spec, full_spec, bias_spec,
        ],
        out_specs=row_spec,
        out_shape=jax.ShapeDtypeStruct((_N, _D), jnp.float32),
    )(eps, x, Sa, Sb, Wt1, bt1, Wt2, bt2)


def kernel(x, edge_index, eps, W1, b1, W2, b2):
    src = edge_index[0]
    dst = edge_index[1]
    pad = jnp.full((_EP - _E,), _N, dtype=jnp.int32)
    src_p = jnp.concatenate([src, pad])
    dst_p = jnp.concatenate([dst, pad])
    x_pad = jnp.concatenate([x, jnp.zeros((_NP - _N, _D), x.dtype)], axis=0)

    zeros_acc = jnp.zeros((_NP, _D), jnp.float32)

    deg2d = _tc_degree(src)
    deg_col = deg2d.reshape(-1, 1)[:_NP]
    y = _tc_scale(x_pad, deg_col)
    src2d = src_p.reshape(_NCHUNKS, _CHUNK)
    s2 = _sc_segsum_kernel()(src2d, dst_p, y, zeros_acc)
    Sa = s2[:_N]
    Sb = s2[_NP:_NP + _N]

    Wt1 = jnp.zeros((_D, _D), jnp.float32).at[1:, 1:].set(W1.T)
    Wt2 = jnp.zeros((_D, _D), jnp.float32).at[1:, 1:].set(W2.T)
    bt1 = jnp.concatenate([jnp.zeros((1,), jnp.float32), b1]).reshape(1, _D)
    bt2 = jnp.concatenate([jnp.zeros((1,), jnp.float32), b2]).reshape(1, _D)
    eps_arr = jnp.asarray(eps, jnp.float32).reshape(1, 1)

    return _tc_dense(x, Sa, Sb, eps_arr, Wt1, bt1, Wt2, bt2)


# speedup vs baseline: 1.0332x; 1.0332x over previous
import functools

import jax
import jax.numpy as jnp
from jax import lax
from jax.experimental import pallas as pl
from jax.experimental.pallas import tpu as pltpu
from jax.experimental.pallas import tpu_sc as plsc

EPSN = 1e-9

_N = 10000
_D = 128
_E = 320000
_NP = 10240
_CHUNK = 128
_NCHUNKS = 2560
_EP = _NCHUNKS * _CHUNK
_H = _D // 2
_ROWS_PER_TILE = _NP // 16
_CHUNKS_C0 = 104
_CHUNKS_C1 = 56
_CHUNKS_MAX = 104


def _sc_mesh():
    return plsc.VectorSubcoreMesh(core_axis_name="c", subcore_axis_name="s")


_HCHUNK = 2000


def _tc_degree_body(src_ref, deg_ref):
    i = pl.program_id(0)

    @pl.when(i == 0)
    def _():
        deg_ref[...] = jnp.zeros_like(deg_ref)

    s = src_ref[0, 0, :]
    hi = jax.lax.shift_right_logical(s, 7)
    lo = jax.lax.bitwise_and(s, 127)
    r = lax.broadcasted_iota(jnp.int32, (_D, _HCHUNK), 0)
    oh_hi_t = (r == hi[None, :]).astype(jnp.bfloat16)
    c = lax.broadcasted_iota(jnp.int32, (_HCHUNK, _D), 1)
    oh_lo = (c == lo[:, None]).astype(jnp.bfloat16)
    deg_ref[...] += jnp.dot(oh_hi_t, oh_lo, preferred_element_type=jnp.float32)


def _tc_degree(src):
    grid = (_E // _HCHUNK,)
    return pl.pallas_call(
        _tc_degree_body,
        grid=grid,
        in_specs=[pl.BlockSpec((1, 1, _HCHUNK), lambda i: (i, 0, 0))],
        out_specs=pl.BlockSpec((_D, _D), lambda i: (0, 0)),
        out_shape=jax.ShapeDtypeStruct((_D, _D), jnp.float32),
    )(src.reshape(_E // _HCHUNK, 1, _HCHUNK))


@functools.cache
def _sc_segsum_kernel():
    return functools.partial(
        pl.kernel,
        mesh=_sc_mesh(),
        out_type=jax.ShapeDtypeStruct((2 * _NP, _D), jnp.float32),
        scratch_types=[
            pltpu.VMEM((_CHUNK,), jnp.int32),
            pltpu.VMEM((_CHUNK,), jnp.int32),
            pltpu.VMEM((_CHUNK,), jnp.int32),
            pltpu.VMEM((_CHUNK,), jnp.int32),
            pltpu.VMEM((_CHUNK, _D), jnp.float32),
            pltpu.VMEM((_CHUNK, _D), jnp.float32),
            pltpu.VMEM_SHARED((_NP, _D), jnp.float32),
            pltpu.SemaphoreType.DMA,
            pltpu.SemaphoreType.DMA,
            pltpu.SemaphoreType.DMA,
            pltpu.SemaphoreType.DMA,
            pltpu.SemaphoreType.DMA,
            pltpu.SemaphoreType.DMA,
        ],
    )(_sc_segsum_body)


def _sc_segsum_body(src_hbm, dst_hbm, y_hbm, zeros_hbm, out_hbm,
                    srci0, srci1, dsti0, dsti1, rows0, rows1, acc_sp,
                    semg0, semg1, semd0, semd1, sems0, sems1):
    c = lax.axis_index("c")
    s = lax.axis_index("s")
    n = jnp.where(c == 0, _CHUNKS_C0, _CHUNKS_C1)
    base = jnp.where(c == 0, s * _CHUNKS_C0, 16 * _CHUNKS_C0 + s * _CHUNKS_C1)

    stripe = pl.ds(s * _ROWS_PER_TILE, _ROWS_PER_TILE)
    pltpu.sync_copy(zeros_hbm.at[stripe], acc_sp.at[stripe])
    plsc.subcore_barrier()

    rows = (rows0, rows1)
    dsti = (dsti0, dsti1)
    srci = (srci0, srci1)
    semg = (semg0, semg1)
    semd = (semd0, semd1)
    sems = (sems0, sems1)

    def idx_load(j, b):
        off = (base + j) * _CHUNK
        pltpu.async_copy(dst_hbm.at[pl.ds(off, _CHUNK)], dsti[b], semd[b])
        pltpu.async_copy(src_hbm.at[pl.ds(off, _CHUNK)], srci[b], sems[b])

    def start_gather(b):
        pltpu.make_async_copy(dst_hbm.at[pl.ds(0, _CHUNK)], dsti[b],
                              semd[b]).wait()
        pltpu.async_copy(y_hbm.at[dsti[b]], rows[b], semg[b])

    def finish_chunk(j, b):
        pltpu.make_async_copy(y_hbm.at[dsti[b]], rows[b], semg[b]).wait()

        @pl.when(j + 2 < n)
        def _():
            idx_load(j + 2, b)

        pltpu.make_async_copy(src_hbm.at[pl.ds(0, _CHUNK)], srci[b],
                              sems[b]).wait()
        pltpu.sync_copy(rows[b], acc_sp.at[srci[b]], add=True)

    idx_load(0, 0)
    idx_load(1, 1)
    start_gather(0)
    npairs = n // 2

    def body(g, carry):
        start_gather(1)
        finish_chunk(g * 2, 0)

        @pl.when(g < npairs - 1)
        def _():
            start_gather(0)

        finish_chunk(g * 2 + 1, 1)
        return carry

    lax.fori_loop(0, npairs, body, 0)
    plsc.subcore_barrier()

    pltpu.sync_copy(acc_sp.at[stripe],
                    out_hbm.at[pl.ds(c * _NP + s * _ROWS_PER_TILE, _ROWS_PER_TILE)])


def _tc_scale_body(x_ref, deg_ref, y_ref):
    deg = deg_ref[...]
    dis = jnp.where(deg > 0, lax.rsqrt(deg), 0.0)
    y_ref[...] = x_ref[...] * dis


def _tc_scale(x_pad, deg_col):
    return pl.pallas_call(
        _tc_scale_body,
        out_shape=jax.ShapeDtypeStruct((_NP, _D), jnp.float32),
    )(x_pad, deg_col)


def _acosh(t):
    return jnp.log(t + jnp.sqrt((t - 1.0) * (t + 1.0)))


def _tc_dense_body(eps_ref, x_ref, sa_ref, sb_ref, w1_ref, b1_ref, w2_ref, b2_ref,
                   o_ref):
    R = x_ref.shape[0]
    col = lax.broadcasted_iota(jnp.int32, (R, _D), 1)
    m = jnp.where(col > 0, 1.0, 0.0)
    e0 = jnp.where(col == 0, 1.0, 0.0)

    def logmap0_s(z):
        zs = z * m
        xn = jnp.sqrt(jnp.clip(jnp.sum(zs * zs, axis=-1, keepdims=True), EPSN, None))
        t = jnp.clip(z[:, :1], 1.0 + 1e-7, None)
        return (_acosh(t) / xn) * zs

    def sinh_cosh(n):
        en = jnp.exp(n)
        inv = 1.0 / en
        return 0.5 * (en - inv), 0.5 * (en + inv)

    def exp_proj(v):
        n = jnp.sqrt(jnp.clip(jnp.sum(v * v, axis=-1, keepdims=True), EPSN, None))
        sh, _ = sinh_cosh(n)
        q = (sh / n) * v
        t = jnp.sqrt(1.0 + jnp.sum(q * q, axis=-1, keepdims=True))
        return q + e0 * t

    eps = eps_ref[0, 0]
    x = x_ref[...]
    S = sa_ref[...] + sb_ref[...]

    sumsq = jnp.sum(S * S, axis=-1, keepdims=True)
    inner = sumsq - 2.0 * (S[:, :1] * S[:, :1])
    denom = jnp.sqrt(jnp.clip(-inner, EPSN, None))
    h = S / denom

    v = (1.0 + eps) * logmap0_s(x) + logmap0_s(h)
    z = exp_proj(v)

    def layer(z, w_ref, b_ref):
        u = logmap0_s(z)
        o = jnp.dot(u, w_ref[...], preferred_element_type=jnp.float32) + b_ref[...]
        z1 = exp_proj(o)
        ua = jnp.tanh(logmap0_s(z1))
        n = jnp.sqrt(jnp.clip(jnp.sum(ua * ua, axis=-1, keepdims=True), EPSN, None))
        sh, ch = sinh_cosh(n)
        return (sh / n) * ua + e0 * ch

    z = layer(z, w1_ref, b1_ref)
    z = layer(z, w2_ref, b2_ref)
    o_ref[...] = z


def _tc_dense(x, Sa, Sb, eps, Wt1, bt1, Wt2, bt2):
    R = 1000
    grid = (_N // R,)
    row_spec = pl.BlockSpec((R, _D), lambda i: (i, 0))
    full_spec = pl.BlockSpec((_D, _D), lambda i: (0, 0))
    bias_spec = pl.BlockSpec((1, _D), lambda i: (0, 0))
    return pl.pallas_call(
        _tc_dense_body,
        grid=grid,
        in_specs=[
            pl.BlockSpec(memory_space=pltpu.SMEM),
            row_spec, row_spec, row_spec, full_spec, bias_spec, full_spec, bias_spec,
        ],
        out_specs=row_spec,
        out_shape=jax.ShapeDtypeStruct((_N, _D), jnp.float32),
    )(eps, x, Sa, Sb, Wt1, bt1, Wt2, bt2)


def kernel(x, edge_index, eps, W1, b1, W2, b2):
    src = edge_index[0]
    dst = edge_index[1]
    pad = jnp.full((_EP - _E,), _N, dtype=jnp.int32)
    src_p = jnp.concatenate([src, pad])
    dst_p = jnp.concatenate([dst, pad])
    x_pad = jnp.concatenate([x, jnp.zeros((_NP - _N, _D), x.dtype)], axis=0)

    zeros_acc = jnp.zeros((_NP, _D), jnp.float32)

    deg2d = _tc_degree(src)
    deg_col = deg2d.reshape(-1, 1)[:_NP]
    y = _tc_scale(x_pad, deg_col)
    s2 = _sc_segsum_kernel()(src_p, dst_p, y, zeros_acc)
    Sa = s2[:_N]
    Sb = s2[_NP:_NP + _N]

    Wt1 = jnp.zeros((_D, _D), jnp.float32).at[1:, 1:].set(W1.T)
    Wt2 = jnp.zeros((_D, _D), jnp.float32).at[1:, 1:].set(W2.T)
    bt1 = jnp.concatenate([jnp.zeros((1,), jnp.float32), b1]).reshape(1, _D)
    bt2 = jnp.concatenate([jnp.zeros((1,), jnp.float32), b2]).reshape(1, _D)
    eps_arr = jnp.asarray(eps, jnp.float32).reshape(1, 1)

    return _tc_dense(x, Sa, Sb, eps_arr, Wt1, bt1, Wt2, bt2)


# speedup vs baseline: 1.5438x; 1.4942x over previous
import functools

import jax
import jax.numpy as jnp
from jax import lax
from jax.experimental import pallas as pl
from jax.experimental.pallas import tpu as pltpu
from jax.experimental.pallas import tpu_sc as plsc

EPSN = 1e-9

_N = 10000
_D = 128
_E = 320000
_NP = 10240
_CHUNK = 128
_NCHUNKS = 2528
_EP = _NCHUNKS * _CHUNK
_H = _D // 2
_ROWS_PER_TILE = _NP // 16
_CHUNKS_C0 = 103
_CHUNKS_C1 = 55


def _sc_mesh():
    return plsc.VectorSubcoreMesh(core_axis_name="c", subcore_axis_name="s")


_HCHUNK = 2000


def _tc_degree_body(src_ref, deg_ref):
    i = pl.program_id(0)

    @pl.when(i == 0)
    def _():
        deg_ref[...] = jnp.zeros_like(deg_ref)

    s = src_ref[0, 0, :]
    hi = jax.lax.shift_right_logical(s, 7)
    lo = jax.lax.bitwise_and(s, 127)
    r = lax.broadcasted_iota(jnp.int32, (_D, _HCHUNK), 0)
    oh_hi_t = (r == hi[None, :]).astype(jnp.bfloat16)
    c = lax.broadcasted_iota(jnp.int32, (_HCHUNK, _D), 1)
    oh_lo = (c == lo[:, None]).astype(jnp.bfloat16)
    deg_ref[...] += jnp.dot(oh_hi_t, oh_lo, preferred_element_type=jnp.float32)


def _tc_degree(src):
    grid = (_E // _HCHUNK,)
    return pl.pallas_call(
        _tc_degree_body,
        grid=grid,
        in_specs=[pl.BlockSpec((1, 1, _HCHUNK), lambda i: (i, 0, 0))],
        out_specs=pl.BlockSpec((_D, _D), lambda i: (0, 0)),
        out_shape=jax.ShapeDtypeStruct((_D, _D), jnp.float32),
    )(src.reshape(_E // _HCHUNK, 1, _HCHUNK))


@functools.cache
def _sc_segsum_kernel():
    return functools.partial(
        pl.kernel,
        mesh=_sc_mesh(),
        out_type=jax.ShapeDtypeStruct((2 * _NP, _D), jnp.float32),
        scratch_types=[
            pltpu.VMEM((_CHUNK,), jnp.int32),
            pltpu.VMEM((_CHUNK,), jnp.int32),
            pltpu.VMEM((_CHUNK,), jnp.int32),
            pltpu.VMEM((_CHUNK,), jnp.int32),
            pltpu.VMEM((_CHUNK, _D), jnp.float32),
            pltpu.VMEM((_CHUNK, _D), jnp.float32),
            pltpu.VMEM_SHARED((_NP, _D), jnp.float32),
            pltpu.SemaphoreType.DMA,
            pltpu.SemaphoreType.DMA,
            pltpu.SemaphoreType.DMA,
            pltpu.SemaphoreType.DMA,
            pltpu.SemaphoreType.DMA,
            pltpu.SemaphoreType.DMA,
        ],
    )(_sc_segsum_body)


def _sc_segsum_body(src_hbm, dst_hbm, y_hbm, zeros_hbm, out_hbm,
                    srci0, srci1, dsti0, dsti1, rows0, rows1, acc_sp,
                    semg0, semg1, semd0, semd1, sems0, sems1):
    c = lax.axis_index("c")
    s = lax.axis_index("s")
    n = jnp.where(c == 0, _CHUNKS_C0, _CHUNKS_C1)
    base = jnp.where(c == 0, s * _CHUNKS_C0, 16 * _CHUNKS_C0 + s * _CHUNKS_C1)

    stripe = pl.ds(s * _ROWS_PER_TILE, _ROWS_PER_TILE)
    pltpu.sync_copy(zeros_hbm.at[stripe], acc_sp.at[stripe])
    plsc.subcore_barrier()

    rows = (rows0, rows1)
    dsti = (dsti0, dsti1)
    srci = (srci0, srci1)
    semg = (semg0, semg1)
    semd = (semd0, semd1)
    sems = (sems0, sems1)

    def idx_load(j, b):
        off = (base + jnp.minimum(j, n - 1)) * _CHUNK
        pltpu.async_copy(dst_hbm.at[pl.ds(off, _CHUNK)], dsti[b], semd[b])
        pltpu.async_copy(src_hbm.at[pl.ds(off, _CHUNK)], srci[b], sems[b])

    def start_gather(b):
        pltpu.make_async_copy(dst_hbm.at[pl.ds(0, _CHUNK)], dsti[b],
                              semd[b]).wait()
        pltpu.async_copy(y_hbm.at[dsti[b]], rows[b], semg[b])

    def finish_chunk(j, b, prefetch=True):
        pltpu.make_async_copy(y_hbm.at[dsti[b]], rows[b], semg[b]).wait()
        pltpu.make_async_copy(src_hbm.at[pl.ds(0, _CHUNK)], srci[b],
                              sems[b]).wait()
        pltpu.sync_copy(rows[b], acc_sp.at[srci[b]], add=True)
        if prefetch:
            idx_load(j + 2, b)

    idx_load(0, 0)
    idx_load(1, 1)
    start_gather(0)

    def body(g, carry):
        start_gather(1)
        finish_chunk(g * 2, 0)
        start_gather(0)
        finish_chunk(g * 2 + 1, 1)
        return carry

    lax.fori_loop(0, (n - 1) // 2, body, 0)
    finish_chunk(n - 1, 0, prefetch=False)
    pltpu.make_async_copy(dst_hbm.at[pl.ds(0, _CHUNK)], dsti[1], semd[1]).wait()
    pltpu.make_async_copy(src_hbm.at[pl.ds(0, _CHUNK)], srci[1], sems[1]).wait()
    plsc.subcore_barrier()

    pltpu.sync_copy(acc_sp.at[stripe],
                    out_hbm.at[pl.ds(c * _NP + s * _ROWS_PER_TILE, _ROWS_PER_TILE)])


def _tc_scale_body(x_ref, deg_ref, y_ref):
    deg = deg_ref[...]
    dis = jnp.where(deg > 0, lax.rsqrt(deg), 0.0)
    y_ref[...] = x_ref[...] * dis


def _tc_scale(x_pad, deg_col):
    return pl.pallas_call(
        _tc_scale_body,
        out_shape=jax.ShapeDtypeStruct((_NP, _D), jnp.float32),
    )(x_pad, deg_col)


def _acosh(t):
    return jnp.log(t + jnp.sqrt((t - 1.0) * (t + 1.0)))


def _tc_dense_body(eps_ref, x_ref, sa_ref, sb_ref, w1_ref, b1_ref, w2_ref, b2_ref,
                   o_ref):
    R = x_ref.shape[0]
    col = lax.broadcasted_iota(jnp.int32, (R, _D), 1)
    m = jnp.where(col > 0, 1.0, 0.0)
    e0 = jnp.where(col == 0, 1.0, 0.0)

    def logmap0_s(z):
        zs = z * m
        xn = jnp.sqrt(jnp.clip(jnp.sum(zs * zs, axis=-1, keepdims=True), EPSN, None))
        t = jnp.clip(z[:, :1], 1.0 + 1e-7, None)
        return (_acosh(t) / xn) * zs

    def sinh_cosh(n):
        en = jnp.exp(n)
        inv = 1.0 / en
        return 0.5 * (en - inv), 0.5 * (en + inv)

    def exp_proj(v):
        n = jnp.sqrt(jnp.clip(jnp.sum(v * v, axis=-1, keepdims=True), EPSN, None))
        sh, _ = sinh_cosh(n)
        q = (sh / n) * v
        t = jnp.sqrt(1.0 + jnp.sum(q * q, axis=-1, keepdims=True))
        return q + e0 * t

    eps = eps_ref[0, 0]
    x = x_ref[...]
    S = sa_ref[...] + sb_ref[...]

    sumsq = jnp.sum(S * S, axis=-1, keepdims=True)
    inner = sumsq - 2.0 * (S[:, :1] * S[:, :1])
    denom = jnp.sqrt(jnp.clip(-inner, EPSN, None))
    h = S / denom

    v = (1.0 + eps) * logmap0_s(x) + logmap0_s(h)
    z = exp_proj(v)

    def layer(z, w_ref, b_ref):
        u = logmap0_s(z)
        o = jnp.dot(u, w_ref[...], preferred_element_type=jnp.float32) + b_ref[...]
        z1 = exp_proj(o)
        ua = jnp.tanh(logmap0_s(z1))
        n = jnp.sqrt(jnp.clip(jnp.sum(ua * ua, axis=-1, keepdims=True), EPSN, None))
        sh, ch = sinh_cosh(n)
        return (sh / n) * ua + e0 * ch

    z = layer(z, w1_ref, b1_ref)
    z = layer(z, w2_ref, b2_ref)
    o_ref[...] = z


def _tc_dense(x, Sa, Sb, eps, Wt1, bt1, Wt2, bt2):
    R = 1000
    grid = (_N // R,)
    row_spec = pl.BlockSpec((R, _D), lambda i: (i, 0))
    full_spec = pl.BlockSpec((_D, _D), lambda i: (0, 0))
    bias_spec = pl.BlockSpec((1, _D), lambda i: (0, 0))
    return pl.pallas_call(
        _tc_dense_body,
        grid=grid,
        in_specs=[
            pl.BlockSpec(memory_space=pltpu.SMEM),
            row_spec, row_spec, row_spec, full_spec, bias_spec, full_spec, bias_spec,
        ],
        out_specs=row_spec,
        out_shape=jax.ShapeDtypeStruct((_N, _D), jnp.float32),
    )(eps, x, Sa, Sb, Wt1, bt1, Wt2, bt2)


def kernel(x, edge_index, eps, W1, b1, W2, b2):
    src = edge_index[0]
    dst = edge_index[1]
    pad = jnp.full((_EP - _E,), _N, dtype=jnp.int32)
    src_p = jnp.concatenate([src, pad])
    dst_p = jnp.concatenate([dst, pad])
    x_pad = jnp.concatenate([x, jnp.zeros((_NP - _N, _D), x.dtype)], axis=0)

    zeros_acc = jnp.zeros((_NP, _D), jnp.float32)

    deg2d = _tc_degree(src)
    deg_col = deg2d.reshape(-1, 1)[:_NP]
    y = _tc_scale(x_pad, deg_col)
    s2 = _sc_segsum_kernel()(src_p, dst_p, y, zeros_acc)
    Sa = s2[:_N]
    Sb = s2[_NP:_NP + _N]

    Wt1 = jnp.zeros((_D, _D), jnp.float32).at[1:, 1:].set(W1.T)
    Wt2 = jnp.zeros((_D, _D), jnp.float32).at[1:, 1:].set(W2.T)
    bt1 = jnp.concatenate([jnp.zeros((1,), jnp.float32), b1]).reshape(1, _D)
    bt2 = jnp.concatenate([jnp.zeros((1,), jnp.float32), b2]).reshape(1, _D)
    eps_arr = jnp.asarray(eps, jnp.float32).reshape(1, 1)

    return _tc_dense(x, Sa, Sb, eps_arr, Wt1, bt1, Wt2, bt2)


# speedup vs baseline: 1.5690x; 1.0163x over previous
import functools

import jax
import jax.numpy as jnp
from jax import lax
from jax.experimental import pallas as pl
from jax.experimental.pallas import tpu as pltpu
from jax.experimental.pallas import tpu_sc as plsc

EPSN = 1e-9

_N = 10000
_D = 128
_E = 320000
_NP = 10240
_CHUNK = 128
_NCHUNKS = 2528
_EP = _NCHUNKS * _CHUNK
_H = _D // 2
_ROWS_PER_TILE = _NP // 16
_CHUNKS_C0 = 111
_CHUNKS_C1 = 47


def _sc_mesh():
    return plsc.VectorSubcoreMesh(core_axis_name="c", subcore_axis_name="s")


_HCHUNK = 2000


def _tc_degree_body(src_ref, deg_ref):
    i = pl.program_id(0)

    @pl.when(i == 0)
    def _():
        deg_ref[...] = jnp.zeros_like(deg_ref)

    s = src_ref[0, 0, :]
    hi = jax.lax.shift_right_logical(s, 7)
    lo = jax.lax.bitwise_and(s, 127)
    r = lax.broadcasted_iota(jnp.int32, (_D, _HCHUNK), 0)
    oh_hi_t = (r == hi[None, :]).astype(jnp.bfloat16)
    c = lax.broadcasted_iota(jnp.int32, (_HCHUNK, _D), 1)
    oh_lo = (c == lo[:, None]).astype(jnp.bfloat16)
    deg_ref[...] += jnp.dot(oh_hi_t, oh_lo, preferred_element_type=jnp.float32)


def _tc_degree(src):
    grid = (_E // _HCHUNK,)
    return pl.pallas_call(
        _tc_degree_body,
        grid=grid,
        in_specs=[pl.BlockSpec((1, 1, _HCHUNK), lambda i: (i, 0, 0))],
        out_specs=pl.BlockSpec((_D, _D), lambda i: (0, 0)),
        out_shape=jax.ShapeDtypeStruct((_D, _D), jnp.float32),
    )(src.reshape(_E // _HCHUNK, 1, _HCHUNK))


@functools.cache
def _sc_segsum_kernel():
    return functools.partial(
        pl.kernel,
        mesh=_sc_mesh(),
        out_type=jax.ShapeDtypeStruct((2 * _NP, _D), jnp.float32),
        scratch_types=[
            pltpu.VMEM((_CHUNK,), jnp.int32),
            pltpu.VMEM((_CHUNK,), jnp.int32),
            pltpu.VMEM((_CHUNK,), jnp.int32),
            pltpu.VMEM((_CHUNK,), jnp.int32),
            pltpu.VMEM((_CHUNK, _D), jnp.float32),
            pltpu.VMEM((_CHUNK, _D), jnp.float32),
            pltpu.VMEM_SHARED((_NP, _D), jnp.float32),
            pltpu.SemaphoreType.DMA,
            pltpu.SemaphoreType.DMA,
            pltpu.SemaphoreType.DMA,
            pltpu.SemaphoreType.DMA,
            pltpu.SemaphoreType.DMA,
            pltpu.SemaphoreType.DMA,
        ],
    )(_sc_segsum_body)


def _sc_segsum_body(src_hbm, dst_hbm, y_hbm, zeros_hbm, out_hbm,
                    srci0, srci1, dsti0, dsti1, rows0, rows1, acc_sp,
                    semg0, semg1, semd0, semd1, sems0, sems1):
    c = lax.axis_index("c")
    s = lax.axis_index("s")
    n = jnp.where(c == 0, _CHUNKS_C0, _CHUNKS_C1)
    base = jnp.where(c == 0, s * _CHUNKS_C0, 16 * _CHUNKS_C0 + s * _CHUNKS_C1)

    stripe = pl.ds(s * _ROWS_PER_TILE, _ROWS_PER_TILE)
    pltpu.sync_copy(zeros_hbm.at[stripe], acc_sp.at[stripe])
    plsc.subcore_barrier()

    rows = (rows0, rows1)
    dsti = (dsti0, dsti1)
    srci = (srci0, srci1)
    semg = (semg0, semg1)
    semd = (semd0, semd1)
    sems = (sems0, sems1)

    def idx_load(j, b):
        off = (base + jnp.minimum(j, n - 1)) * _CHUNK
        pltpu.async_copy(dst_hbm.at[pl.ds(off, _CHUNK)], dsti[b], semd[b])
        pltpu.async_copy(src_hbm.at[pl.ds(off, _CHUNK)], srci[b], sems[b])

    def start_gather(b):
        pltpu.make_async_copy(dst_hbm.at[pl.ds(0, _CHUNK)], dsti[b],
                              semd[b]).wait()
        pltpu.async_copy(y_hbm.at[dsti[b]], rows[b], semg[b])

    def finish_chunk(j, b, prefetch=True):
        pltpu.make_async_copy(y_hbm.at[dsti[b]], rows[b], semg[b]).wait()
        pltpu.make_async_copy(src_hbm.at[pl.ds(0, _CHUNK)], srci[b],
                              sems[b]).wait()
        pltpu.sync_copy(rows[b], acc_sp.at[srci[b]], add=True)
        if prefetch:
            idx_load(j + 2, b)

    idx_load(0, 0)
    idx_load(1, 1)
    start_gather(0)

    def body(g, carry):
        start_gather(1)
        finish_chunk(g * 2, 0)
        start_gather(0)
        finish_chunk(g * 2 + 1, 1)
        return carry

    lax.fori_loop(0, (n - 1) // 2, body, 0)
    finish_chunk(n - 1, 0, prefetch=False)
    pltpu.make_async_copy(dst_hbm.at[pl.ds(0, _CHUNK)], dsti[1], semd[1]).wait()
    pltpu.make_async_copy(src_hbm.at[pl.ds(0, _CHUNK)], srci[1], sems[1]).wait()
    plsc.subcore_barrier()

    pltpu.sync_copy(acc_sp.at[stripe],
                    out_hbm.at[pl.ds(c * _NP + s * _ROWS_PER_TILE, _ROWS_PER_TILE)])


def _tc_scale_body(x_ref, deg_ref, y_ref):
    deg = deg_ref[...]
    dis = jnp.where(deg > 0, lax.rsqrt(deg), 0.0)
    y_ref[...] = x_ref[...] * dis


def _tc_scale(x_pad, deg_col):
    return pl.pallas_call(
        _tc_scale_body,
        out_shape=jax.ShapeDtypeStruct((_NP, _D), jnp.float32),
    )(x_pad, deg_col)


def _acosh(t):
    return jnp.log(t + jnp.sqrt((t - 1.0) * (t + 1.0)))


def _tc_dense_body(eps_ref, x_ref, sa_ref, sb_ref, w1_ref, b1_ref, w2_ref, b2_ref,
                   o_ref):
    R = x_ref.shape[0]
    col = lax.broadcasted_iota(jnp.int32, (R, _D), 1)
    m = jnp.where(col > 0, 1.0, 0.0)
    e0 = jnp.where(col == 0, 1.0, 0.0)

    def logmap0_s(z):
        zs = z * m
        xn = jnp.sqrt(jnp.clip(jnp.sum(zs * zs, axis=-1, keepdims=True), EPSN, None))
        t = jnp.clip(z[:, :1], 1.0 + 1e-7, None)
        return (_acosh(t) / xn) * zs

    def sinh_cosh(n):
        en = jnp.exp(n)
        inv = 1.0 / en
        return 0.5 * (en - inv), 0.5 * (en + inv)

    def exp_proj(v):
        n = jnp.sqrt(jnp.clip(jnp.sum(v * v, axis=-1, keepdims=True), EPSN, None))
        sh, _ = sinh_cosh(n)
        q = (sh / n) * v
        t = jnp.sqrt(1.0 + jnp.sum(q * q, axis=-1, keepdims=True))
        return q + e0 * t

    eps = eps_ref[0, 0]
    x = x_ref[...]
    S = sa_ref[...] + sb_ref[...]

    sumsq = jnp.sum(S * S, axis=-1, keepdims=True)
    inner = sumsq - 2.0 * (S[:, :1] * S[:, :1])
    denom = jnp.sqrt(jnp.clip(-inner, EPSN, None))
    h = S / denom

    v = (1.0 + eps) * logmap0_s(x) + logmap0_s(h)
    z = exp_proj(v)

    def layer(z, w_ref, b_ref):
        u = logmap0_s(z)
        o = jnp.dot(u, w_ref[...], preferred_element_type=jnp.float32) + b_ref[...]
        z1 = exp_proj(o)
        ua = jnp.tanh(logmap0_s(z1))
        n = jnp.sqrt(jnp.clip(jnp.sum(ua * ua, axis=-1, keepdims=True), EPSN, None))
        sh, ch = sinh_cosh(n)
        return (sh / n) * ua + e0 * ch

    z = layer(z, w1_ref, b1_ref)
    z = layer(z, w2_ref, b2_ref)
    o_ref[...] = z


def _tc_dense(x, Sa, Sb, eps, Wt1, bt1, Wt2, bt2):
    R = 1000
    grid = (_N // R,)
    row_spec = pl.BlockSpec((R, _D), lambda i: (i, 0))
    full_spec = pl.BlockSpec((_D, _D), lambda i: (0, 0))
    bias_spec = pl.BlockSpec((1, _D), lambda i: (0, 0))
    return pl.pallas_call(
        _tc_dense_body,
        grid=grid,
        in_specs=[
            pl.BlockSpec(memory_space=pltpu.SMEM),
            row_spec, row_spec, row_spec, full_spec, bias_spec, full_spec, bias_spec,
        ],
        out_specs=row_spec,
        out_shape=jax.ShapeDtypeStruct((_N, _D), jnp.float32),
    )(eps, x, Sa, Sb, Wt1, bt1, Wt2, bt2)


def kernel(x, edge_index, eps, W1, b1, W2, b2):
    src = edge_index[0]
    dst = edge_index[1]
    pad = jnp.full((_EP - _E,), _N, dtype=jnp.int32)
    src_p = jnp.concatenate([src, pad])
    dst_p = jnp.concatenate([dst, pad])
    x_pad = jnp.concatenate([x, jnp.zeros((_NP - _N, _D), x.dtype)], axis=0)

    zeros_acc = jnp.zeros((_NP, _D), jnp.float32)

    deg2d = _tc_degree(src)
    deg_col = deg2d.reshape(-1, 1)[:_NP]
    y = _tc_scale(x_pad, deg_col)
    s2 = _sc_segsum_kernel()(src_p, dst_p, y, zeros_acc)
    Sa = s2[:_N]
    Sb = s2[_NP:_NP + _N]

    Wt1 = jnp.zeros((_D, _D), jnp.float32).at[1:, 1:].set(W1.T)
    Wt2 = jnp.zeros((_D, _D), jnp.float32).at[1:, 1:].set(W2.T)
    bt1 = jnp.concatenate([jnp.zeros((1,), jnp.float32), b1]).reshape(1, _D)
    bt2 = jnp.concatenate([jnp.zeros((1,), jnp.float32), b2]).reshape(1, _D)
    eps_arr = jnp.asarray(eps, jnp.float32).reshape(1, 1)

    return _tc_dense(x, Sa, Sb, eps_arr, Wt1, bt1, Wt2, bt2)


# speedup vs baseline: 1.5912x; 1.0141x over previous
import functools

import jax
import jax.numpy as jnp
from jax import lax
from jax.experimental import pallas as pl
from jax.experimental.pallas import tpu as pltpu
from jax.experimental.pallas import tpu_sc as plsc

EPSN = 1e-9

_N = 10000
_D = 128
_E = 320000
_NP = 10240
_CHUNK = 128
_NCHUNKS = 2528
_EP = _NCHUNKS * _CHUNK
_H = _D // 2
_ROWS_PER_TILE = _NP // 16
_CHUNKS_C0 = 117
_CHUNKS_C1 = 41


def _sc_mesh():
    return plsc.VectorSubcoreMesh(core_axis_name="c", subcore_axis_name="s")


_HCHUNK = 2000


def _tc_degree_body(src_ref, deg_ref):
    i = pl.program_id(0)

    @pl.when(i == 0)
    def _():
        deg_ref[...] = jnp.zeros_like(deg_ref)

    s = src_ref[0, 0, :]
    hi = jax.lax.shift_right_logical(s, 7)
    lo = jax.lax.bitwise_and(s, 127)
    r = lax.broadcasted_iota(jnp.int32, (_D, _HCHUNK), 0)
    oh_hi_t = (r == hi[None, :]).astype(jnp.bfloat16)
    c = lax.broadcasted_iota(jnp.int32, (_HCHUNK, _D), 1)
    oh_lo = (c == lo[:, None]).astype(jnp.bfloat16)
    deg_ref[...] += jnp.dot(oh_hi_t, oh_lo, preferred_element_type=jnp.float32)


def _tc_degree(src):
    grid = (_E // _HCHUNK,)
    return pl.pallas_call(
        _tc_degree_body,
        grid=grid,
        in_specs=[pl.BlockSpec((1, 1, _HCHUNK), lambda i: (i, 0, 0))],
        out_specs=pl.BlockSpec((_D, _D), lambda i: (0, 0)),
        out_shape=jax.ShapeDtypeStruct((_D, _D), jnp.float32),
    )(src.reshape(_E // _HCHUNK, 1, _HCHUNK))


@functools.cache
def _sc_segsum_kernel():
    return functools.partial(
        pl.kernel,
        mesh=_sc_mesh(),
        out_type=jax.ShapeDtypeStruct((2 * _NP, _D), jnp.float32),
        scratch_types=[
            pltpu.VMEM((_CHUNK,), jnp.int32),
            pltpu.VMEM((_CHUNK,), jnp.int32),
            pltpu.VMEM((_CHUNK,), jnp.int32),
            pltpu.VMEM((_CHUNK,), jnp.int32),
            pltpu.VMEM((_CHUNK, _D), jnp.float32),
            pltpu.VMEM((_CHUNK, _D), jnp.float32),
            pltpu.VMEM_SHARED((_NP, _D), jnp.float32),
            pltpu.SemaphoreType.DMA,
            pltpu.SemaphoreType.DMA,
            pltpu.SemaphoreType.DMA,
            pltpu.SemaphoreType.DMA,
            pltpu.SemaphoreType.DMA,
            pltpu.SemaphoreType.DMA,
        ],
    )(_sc_segsum_body)


def _sc_segsum_body(src_hbm, dst_hbm, y_hbm, zeros_hbm, out_hbm,
                    srci0, srci1, dsti0, dsti1, rows0, rows1, acc_sp,
                    semg0, semg1, semd0, semd1, sems0, sems1):
    c = lax.axis_index("c")
    s = lax.axis_index("s")
    n = jnp.where(c == 0, _CHUNKS_C0, _CHUNKS_C1)
    base = jnp.where(c == 0, s * _CHUNKS_C0, 16 * _CHUNKS_C0 + s * _CHUNKS_C1)

    stripe = pl.ds(s * _ROWS_PER_TILE, _ROWS_PER_TILE)
    pltpu.sync_copy(zeros_hbm.at[stripe], acc_sp.at[stripe])
    plsc.subcore_barrier()

    rows = (rows0, rows1)
    dsti = (dsti0, dsti1)
    srci = (srci0, srci1)
    semg = (semg0, semg1)
    semd = (semd0, semd1)
    sems = (sems0, sems1)

    def idx_load(j, b):
        off = (base + jnp.minimum(j, n - 1)) * _CHUNK
        pltpu.async_copy(dst_hbm.at[pl.ds(off, _CHUNK)], dsti[b], semd[b])
        pltpu.async_copy(src_hbm.at[pl.ds(off, _CHUNK)], srci[b], sems[b])

    def start_gather(b):
        pltpu.make_async_copy(dst_hbm.at[pl.ds(0, _CHUNK)], dsti[b],
                              semd[b]).wait()
        pltpu.async_copy(y_hbm.at[dsti[b]], rows[b], semg[b])

    def finish_chunk(j, b, prefetch=True):
        pltpu.make_async_copy(y_hbm.at[dsti[b]], rows[b], semg[b]).wait()
        pltpu.make_async_copy(src_hbm.at[pl.ds(0, _CHUNK)], srci[b],
                              sems[b]).wait()
        pltpu.sync_copy(rows[b], acc_sp.at[srci[b]], add=True)
        if prefetch:
            idx_load(j + 2, b)

    idx_load(0, 0)
    idx_load(1, 1)
    start_gather(0)

    def body(g, carry):
        start_gather(1)
        finish_chunk(g * 2, 0)
        start_gather(0)
        finish_chunk(g * 2 + 1, 1)
        return carry

    lax.fori_loop(0, (n - 1) // 2, body, 0)
    finish_chunk(n - 1, 0, prefetch=False)
    pltpu.make_async_copy(dst_hbm.at[pl.ds(0, _CHUNK)], dsti[1], semd[1]).wait()
    pltpu.make_async_copy(src_hbm.at[pl.ds(0, _CHUNK)], srci[1], sems[1]).wait()
    plsc.subcore_barrier()

    pltpu.sync_copy(acc_sp.at[stripe],
                    out_hbm.at[pl.ds(c * _NP + s * _ROWS_PER_TILE, _ROWS_PER_TILE)])


def _tc_scale_body(x_ref, deg_ref, y_ref):
    deg = deg_ref[...]
    dis = jnp.where(deg > 0, lax.rsqrt(deg), 0.0)
    y_ref[...] = x_ref[...] * dis


def _tc_scale(x_pad, deg_col):
    return pl.pallas_call(
        _tc_scale_body,
        out_shape=jax.ShapeDtypeStruct((_NP, _D), jnp.float32),
    )(x_pad, deg_col)


def _acosh(t):
    return jnp.log(t + jnp.sqrt((t - 1.0) * (t + 1.0)))


def _tc_dense_body(eps_ref, x_ref, sa_ref, sb_ref, w1_ref, b1_ref, w2_ref, b2_ref,
                   o_ref):
    R = x_ref.shape[0]
    col = lax.broadcasted_iota(jnp.int32, (R, _D), 1)
    m = jnp.where(col > 0, 1.0, 0.0)
    e0 = jnp.where(col == 0, 1.0, 0.0)

    def logmap0_s(z):
        zs = z * m
        xn = jnp.sqrt(jnp.clip(jnp.sum(zs * zs, axis=-1, keepdims=True), EPSN, None))
        t = jnp.clip(z[:, :1], 1.0 + 1e-7, None)
        return (_acosh(t) / xn) * zs

    def sinh_cosh(n):
        en = jnp.exp(n)
        inv = 1.0 / en
        return 0.5 * (en - inv), 0.5 * (en + inv)

    def exp_proj(v):
        n = jnp.sqrt(jnp.clip(jnp.sum(v * v, axis=-1, keepdims=True), EPSN, None))
        sh, _ = sinh_cosh(n)
        q = (sh / n) * v
        t = jnp.sqrt(1.0 + jnp.sum(q * q, axis=-1, keepdims=True))
        return q + e0 * t

    eps = eps_ref[0, 0]
    x = x_ref[...]
    S = sa_ref[...] + sb_ref[...]

    sumsq = jnp.sum(S * S, axis=-1, keepdims=True)
    inner = sumsq - 2.0 * (S[:, :1] * S[:, :1])
    denom = jnp.sqrt(jnp.clip(-inner, EPSN, None))
    h = S / denom

    v = (1.0 + eps) * logmap0_s(x) + logmap0_s(h)
    z = exp_proj(v)

    def layer(z, w_ref, b_ref):
        u = logmap0_s(z)
        o = jnp.dot(u, w_ref[...], preferred_element_type=jnp.float32) + b_ref[...]
        z1 = exp_proj(o)
        ua = jnp.tanh(logmap0_s(z1))
        n = jnp.sqrt(jnp.clip(jnp.sum(ua * ua, axis=-1, keepdims=True), EPSN, None))
        sh, ch = sinh_cosh(n)
        return (sh / n) * ua + e0 * ch

    z = layer(z, w1_ref, b1_ref)
    z = layer(z, w2_ref, b2_ref)
    o_ref[...] = z


def _tc_dense(x, Sa, Sb, eps, Wt1, bt1, Wt2, bt2):
    R = 1000
    grid = (_N // R,)
    row_spec = pl.BlockSpec((R, _D), lambda i: (i, 0))
    full_spec = pl.BlockSpec((_D, _D), lambda i: (0, 0))
    bias_spec = pl.BlockSpec((1, _D), lambda i: (0, 0))
    return pl.pallas_call(
        _tc_dense_body,
        grid=grid,
        in_specs=[
            pl.BlockSpec(memory_space=pltpu.SMEM),
            row_spec, row_spec, row_spec, full_spec, bias_spec, full_spec, bias_spec,
        ],
        out_specs=row_spec,
        out_shape=jax.ShapeDtypeStruct((_N, _D), jnp.float32),
    )(eps, x, Sa, Sb, Wt1, bt1, Wt2, bt2)


def kernel(x, edge_index, eps, W1, b1, W2, b2):
    src = edge_index[0]
    dst = edge_index[1]
    pad = jnp.full((_EP - _E,), _N, dtype=jnp.int32)
    src_p = jnp.concatenate([src, pad])
    dst_p = jnp.concatenate([dst, pad])
    x_pad = jnp.concatenate([x, jnp.zeros((_NP - _N, _D), x.dtype)], axis=0)

    zeros_acc = jnp.zeros((_NP, _D), jnp.float32)

    deg2d = _tc_degree(src)
    deg_col = deg2d.reshape(-1, 1)[:_NP]
    y = _tc_scale(x_pad, deg_col)
    s2 = _sc_segsum_kernel()(src_p, dst_p, y, zeros_acc)
    Sa = s2[:_N]
    Sb = s2[_NP:_NP + _N]

    Wt1 = jnp.zeros((_D, _D), jnp.float32).at[1:, 1:].set(W1.T)
    Wt2 = jnp.zeros((_D, _D), jnp.float32).at[1:, 1:].set(W2.T)
    bt1 = jnp.concatenate([jnp.zeros((1,), jnp.float32), b1]).reshape(1, _D)
    bt2 = jnp.concatenate([jnp.zeros((1,), jnp.float32), b2]).reshape(1, _D)
    eps_arr = jnp.asarray(eps, jnp.float32).reshape(1, 1)

    return _tc_dense(x, Sa, Sb, eps_arr, Wt1, bt1, Wt2, bt2)


# speedup vs baseline: 1.6060x; 1.0094x over previous
import functools

import jax
import jax.numpy as jnp
from jax import lax
from jax.experimental import pallas as pl
from jax.experimental.pallas import tpu as pltpu
from jax.experimental.pallas import tpu_sc as plsc

EPSN = 1e-9

_N = 10000
_D = 128
_E = 320000
_NP = 10240
_CHUNK = 128
_NCHUNKS = 2528
_EP = _NCHUNKS * _CHUNK
_H = _D // 2
_ROWS_PER_TILE = _NP // 16
_CHUNKS_C0 = 123
_CHUNKS_C1 = 35


def _sc_mesh():
    return plsc.VectorSubcoreMesh(core_axis_name="c", subcore_axis_name="s")


_HCHUNK = 2000


def _tc_degree_body(src_ref, deg_ref):
    i = pl.program_id(0)

    @pl.when(i == 0)
    def _():
        deg_ref[...] = jnp.zeros_like(deg_ref)

    s = src_ref[0, 0, :]
    hi = jax.lax.shift_right_logical(s, 7)
    lo = jax.lax.bitwise_and(s, 127)
    r = lax.broadcasted_iota(jnp.int32, (_D, _HCHUNK), 0)
    oh_hi_t = (r == hi[None, :]).astype(jnp.bfloat16)
    c = lax.broadcasted_iota(jnp.int32, (_HCHUNK, _D), 1)
    oh_lo = (c == lo[:, None]).astype(jnp.bfloat16)
    deg_ref[...] += jnp.dot(oh_hi_t, oh_lo, preferred_element_type=jnp.float32)


def _tc_degree(src):
    grid = (_E // _HCHUNK,)
    return pl.pallas_call(
        _tc_degree_body,
        grid=grid,
        in_specs=[pl.BlockSpec((1, 1, _HCHUNK), lambda i: (i, 0, 0))],
        out_specs=pl.BlockSpec((_D, _D), lambda i: (0, 0)),
        out_shape=jax.ShapeDtypeStruct((_D, _D), jnp.float32),
    )(src.reshape(_E // _HCHUNK, 1, _HCHUNK))


@functools.cache
def _sc_segsum_kernel():
    return functools.partial(
        pl.kernel,
        mesh=_sc_mesh(),
        out_type=jax.ShapeDtypeStruct((2 * _NP, _D), jnp.float32),
        scratch_types=[
            pltpu.VMEM((_CHUNK,), jnp.int32),
            pltpu.VMEM((_CHUNK,), jnp.int32),
            pltpu.VMEM((_CHUNK,), jnp.int32),
            pltpu.VMEM((_CHUNK,), jnp.int32),
            pltpu.VMEM((_CHUNK, _D), jnp.float32),
            pltpu.VMEM((_CHUNK, _D), jnp.float32),
            pltpu.VMEM_SHARED((_NP, _D), jnp.float32),
            pltpu.SemaphoreType.DMA,
            pltpu.SemaphoreType.DMA,
            pltpu.SemaphoreType.DMA,
            pltpu.SemaphoreType.DMA,
            pltpu.SemaphoreType.DMA,
            pltpu.SemaphoreType.DMA,
        ],
    )(_sc_segsum_body)


def _sc_segsum_body(src_hbm, dst_hbm, y_hbm, zeros_hbm, out_hbm,
                    srci0, srci1, dsti0, dsti1, rows0, rows1, acc_sp,
                    semg0, semg1, semd0, semd1, sems0, sems1):
    c = lax.axis_index("c")
    s = lax.axis_index("s")
    n = jnp.where(c == 0, _CHUNKS_C0, _CHUNKS_C1)
    base = jnp.where(c == 0, s * _CHUNKS_C0, 16 * _CHUNKS_C0 + s * _CHUNKS_C1)

    stripe = pl.ds(s * _ROWS_PER_TILE, _ROWS_PER_TILE)
    pltpu.sync_copy(zeros_hbm.at[stripe], acc_sp.at[stripe])
    plsc.subcore_barrier()

    rows = (rows0, rows1)
    dsti = (dsti0, dsti1)
    srci = (srci0, srci1)
    semg = (semg0, semg1)
    semd = (semd0, semd1)
    sems = (sems0, sems1)

    def idx_load(j, b):
        off = (base + jnp.minimum(j, n - 1)) * _CHUNK
        pltpu.async_copy(dst_hbm.at[pl.ds(off, _CHUNK)], dsti[b], semd[b])
        pltpu.async_copy(src_hbm.at[pl.ds(off, _CHUNK)], srci[b], sems[b])

    def start_gather(b):
        pltpu.make_async_copy(dst_hbm.at[pl.ds(0, _CHUNK)], dsti[b],
                              semd[b]).wait()
        pltpu.async_copy(y_hbm.at[dsti[b]], rows[b], semg[b])

    def finish_chunk(j, b, prefetch=True):
        pltpu.make_async_copy(y_hbm.at[dsti[b]], rows[b], semg[b]).wait()
        pltpu.make_async_copy(src_hbm.at[pl.ds(0, _CHUNK)], srci[b],
                              sems[b]).wait()
        pltpu.sync_copy(rows[b], acc_sp.at[srci[b]], add=True)
        if prefetch:
            idx_load(j + 2, b)

    idx_load(0, 0)
    idx_load(1, 1)
    start_gather(0)

    def body(g, carry):
        start_gather(1)
        finish_chunk(g * 2, 0)
        start_gather(0)
        finish_chunk(g * 2 + 1, 1)
        return carry

    lax.fori_loop(0, (n - 1) // 2, body, 0)
    finish_chunk(n - 1, 0, prefetch=False)
    pltpu.make_async_copy(dst_hbm.at[pl.ds(0, _CHUNK)], dsti[1], semd[1]).wait()
    pltpu.make_async_copy(src_hbm.at[pl.ds(0, _CHUNK)], srci[1], sems[1]).wait()
    plsc.subcore_barrier()

    pltpu.sync_copy(acc_sp.at[stripe],
                    out_hbm.at[pl.ds(c * _NP + s * _ROWS_PER_TILE, _ROWS_PER_TILE)])


def _tc_scale_body(x_ref, deg_ref, y_ref):
    deg = deg_ref[...]
    dis = jnp.where(deg > 0, lax.rsqrt(deg), 0.0)
    y_ref[...] = x_ref[...] * dis


def _tc_scale(x_pad, deg_col):
    return pl.pallas_call(
        _tc_scale_body,
        out_shape=jax.ShapeDtypeStruct((_NP, _D), jnp.float32),
    )(x_pad, deg_col)


def _acosh(t):
    return jnp.log(t + jnp.sqrt((t - 1.0) * (t + 1.0)))


def _tc_dense_body(eps_ref, x_ref, sa_ref, sb_ref, w1_ref, b1_ref, w2_ref, b2_ref,
                   o_ref):
    R = x_ref.shape[0]
    col = lax.broadcasted_iota(jnp.int32, (R, _D), 1)
    m = jnp.where(col > 0, 1.0, 0.0)
    e0 = jnp.where(col == 0, 1.0, 0.0)

    def logmap0_s(z):
        zs = z * m
        xn = jnp.sqrt(jnp.clip(jnp.sum(zs * zs, axis=-1, keepdims=True), EPSN, None))
        t = jnp.clip(z[:, :1], 1.0 + 1e-7, None)
        return (_acosh(t) / xn) * zs

    def sinh_cosh(n):
        en = jnp.exp(n)
        inv = 1.0 / en
        return 0.5 * (en - inv), 0.5 * (en + inv)

    def exp_proj(v):
        n = jnp.sqrt(jnp.clip(jnp.sum(v * v, axis=-1, keepdims=True), EPSN, None))
        sh, _ = sinh_cosh(n)
        q = (sh / n) * v
        t = jnp.sqrt(1.0 + jnp.sum(q * q, axis=-1, keepdims=True))
        return q + e0 * t

    eps = eps_ref[0, 0]
    x = x_ref[...]
    S = sa_ref[...] + sb_ref[...]

    sumsq = jnp.sum(S * S, axis=-1, keepdims=True)
    inner = sumsq - 2.0 * (S[:, :1] * S[:, :1])
    denom = jnp.sqrt(jnp.clip(-inner, EPSN, None))
    h = S / denom

    v = (1.0 + eps) * logmap0_s(x) + logmap0_s(h)
    z = exp_proj(v)

    def layer(z, w_ref, b_ref):
        u = logmap0_s(z)
        o = jnp.dot(u, w_ref[...], preferred_element_type=jnp.float32) + b_ref[...]
        z1 = exp_proj(o)
        ua = jnp.tanh(logmap0_s(z1))
        n = jnp.sqrt(jnp.clip(jnp.sum(ua * ua, axis=-1, keepdims=True), EPSN, None))
        sh, ch = sinh_cosh(n)
        return (sh / n) * ua + e0 * ch

    z = layer(z, w1_ref, b1_ref)
    z = layer(z, w2_ref, b2_ref)
    o_ref[...] = z


def _tc_dense(x, Sa, Sb, eps, Wt1, bt1, Wt2, bt2):
    R = 1000
    grid = (_N // R,)
    row_spec = pl.BlockSpec((R, _D), lambda i: (i, 0))
    full_spec = pl.BlockSpec((_D, _D), lambda i: (0, 0))
    bias_spec = pl.BlockSpec((1, _D), lambda i: (0, 0))
    return pl.pallas_call(
        _tc_dense_body,
        grid=grid,
        in_specs=[
            pl.BlockSpec(memory_space=pltpu.SMEM),
            row_spec, row_spec, row_spec, full_spec, bias_spec, full_spec, bias_spec,
        ],
        out_specs=row_spec,
        out_shape=jax.ShapeDtypeStruct((_N, _D), jnp.float32),
    )(eps, x, Sa, Sb, Wt1, bt1, Wt2, bt2)


def kernel(x, edge_index, eps, W1, b1, W2, b2):
    src = edge_index[0]
    dst = edge_index[1]
    pad = jnp.full((_EP - _E,), _N, dtype=jnp.int32)
    src_p = jnp.concatenate([src, pad])
    dst_p = jnp.concatenate([dst, pad])
    x_pad = jnp.concatenate([x, jnp.zeros((_NP - _N, _D), x.dtype)], axis=0)

    zeros_acc = jnp.zeros((_NP, _D), jnp.float32)

    deg2d = _tc_degree(src)
    deg_col = deg2d.reshape(-1, 1)[:_NP]
    y = _tc_scale(x_pad, deg_col)
    s2 = _sc_segsum_kernel()(src_p, dst_p, y, zeros_acc)
    Sa = s2[:_N]
    Sb = s2[_NP:_NP + _N]

    Wt1 = jnp.zeros((_D, _D), jnp.float32).at[1:, 1:].set(W1.T)
    Wt2 = jnp.zeros((_D, _D), jnp.float32).at[1:, 1:].set(W2.T)
    bt1 = jnp.concatenate([jnp.zeros((1,), jnp.float32), b1]).reshape(1, _D)
    bt2 = jnp.concatenate([jnp.zeros((1,), jnp.float32), b2]).reshape(1, _D)
    eps_arr = jnp.asarray(eps, jnp.float32).reshape(1, 1)

    return _tc_dense(x, Sa, Sb, eps_arr, Wt1, bt1, Wt2, bt2)
